# baseline (speedup 1.0000x reference)
import functools

import jax
import jax.numpy as jnp
from jax.experimental import pallas as pl
from jax.experimental.pallas import tpu as pltpu

K = 8192
N = 8192
D = 32
TN = 256
GRID = N // TN
COMMIT = 0.25


CHUNK = 2048
NCHUNK = K // CHUNK


def _argmin_rowwise(d):
    acc_v = jnp.full((TN,), jnp.inf, jnp.float32)
    acc_i = jnp.zeros((TN,), jnp.int32)
    for c in range(NCHUNK):
        blk = d[:, c * CHUNK:(c + 1) * CHUNK]
        mv = jnp.min(blk, axis=1)
        iota = jax.lax.broadcasted_iota(jnp.int32, (TN, CHUNK), 1)
        mi = jnp.min(jnp.where(blk == mv[:, None], iota, CHUNK), axis=1) + c * CHUNK
        take = mv < acc_v
        acc_i = jnp.where(take, mi, acc_i)
        acc_v = jnp.where(take, mv.astype(jnp.bfloat16).astype(jnp.float32), acc_v)
    return acc_i


def _vq_main(x_ref, w_ref, xsq_ref, wsq_ref, idx_ref, q_ref, counts_ref):
    x = x_ref[...]
    w = w_ref[...]
    mm = jax.lax.dot_general(
        x, w, dimension_numbers=(((1,), (1,)), ((), ())),
        preferred_element_type=jnp.float32)
    d = (xsq_ref[...].reshape(TN, 1) - 2.0 * mm) + wsq_ref[...]
    idx = _argmin_rowwise(d)
    idx_ref[...] = idx.reshape(1, 1, TN)
    oh = (jax.lax.broadcasted_iota(jnp.int32, (TN, K), 1)
          == idx.reshape(TN, 1)).astype(jnp.float32)
    q_ref[...] = jax.lax.dot_general(
        oh, w, dimension_numbers=(((1,), (0,)), ((), ())),
        preferred_element_type=jnp.float32)
    part = jnp.sum(oh, axis=0).reshape(1, K)

    @pl.when(pl.program_id(0) == 0)
    def _init():
        counts_ref[...] = part

    @pl.when(pl.program_id(0) != 0)
    def _acc():
        counts_ref[...] = counts_ref[...] + part


def _vq_stats(x_ref, q_ref, counts_ref, qst_ref, loss_ref, cl_ref, cbl_ref,
              perp_ref, usage_ref):
    x = x_ref[...]
    q = q_ref[...]
    qst_ref[...] = x + (q - x)
    msq = jnp.mean((q - x) ** 2)
    cl_ref[...] = msq.reshape(1, 1)
    cbl_ref[...] = msq.reshape(1, 1)
    loss_ref[...] = (msq + COMMIT * msq).reshape(1, 1)
    counts = counts_ref[...]
    p = counts / jnp.float32(N)
    ent = -jnp.sum(p * jnp.log(p + 1e-10))
    perp_ref[...] = jnp.exp(ent).reshape(1, 1)
    usage_ref[...] = jnp.mean((counts > 0).astype(jnp.float32)).reshape(1, 1)


@jax.jit
def kernel(inputs, weight):
    xsq = jnp.sum(inputs ** 2, axis=1, keepdims=True)
    wsq = jnp.sum(weight ** 2, axis=1)

    idx3, quantized, counts = pl.pallas_call(
        _vq_main,
        grid=(GRID,),
        in_specs=[
            pl.BlockSpec((TN, D), lambda i: (i, 0)),
            pl.BlockSpec((K, D), lambda i: (0, 0)),
            pl.BlockSpec((TN, 1), lambda i: (i, 0)),
            pl.BlockSpec((1, K), lambda i: (0, 0)),
        ],
        out_specs=[
            pl.BlockSpec((1, 1, TN), lambda i: (i, 0, 0)),
            pl.BlockSpec((TN, D), lambda i: (i, 0)),
            pl.BlockSpec((1, K), lambda i: (0, 0)),
        ],
        out_shape=[
            jax.ShapeDtypeStruct((GRID, 1, TN), jnp.int32),
            jax.ShapeDtypeStruct((N, D), jnp.float32),
            jax.ShapeDtypeStruct((1, K), jnp.float32),
        ],
    )(inputs, weight, xsq, wsq.reshape(1, K))

    qst, loss, cl, cbl, perp, usage = pl.pallas_call(
        _vq_stats,
        out_shape=[
            jax.ShapeDtypeStruct((N, D), jnp.float32),
            jax.ShapeDtypeStruct((1, 1), jnp.float32),
            jax.ShapeDtypeStruct((1, 1), jnp.float32),
            jax.ShapeDtypeStruct((1, 1), jnp.float32),
            jax.ShapeDtypeStruct((1, 1), jnp.float32),
            jax.ShapeDtypeStruct((1, 1), jnp.float32),
        ],
    )(inputs, quantized, counts)

    encoding_indices = idx3.reshape(N)
    return (qst, encoding_indices, loss[0, 0], cl[0, 0], cbl[0, 0],
            perp[0, 0], usage[0, 0])


# speedup vs baseline: 1.1219x; 1.1219x over previous
import functools

import jax
import jax.numpy as jnp
from jax import lax
from jax.experimental import pallas as pl
from jax.experimental.pallas import tpu as pltpu
from jax.experimental.pallas import tpu_sc as plsc

K = 8192
N = 8192
D = 32
TN = 256
GRID = N // TN
COMMIT = 0.25

CHUNK = 2048
NCHUNK = K // CHUNK

NC, NS, L = 2, 16, 16
NW = NC * NS
BPW = N // NW


def _argmin_rowwise(d):
    acc_v = jnp.full((TN,), jnp.inf, jnp.float32)
    acc_i = jnp.zeros((TN,), jnp.int32)
    for c in range(NCHUNK):
        blk = d[:, c * CHUNK:(c + 1) * CHUNK]
        mv = jnp.min(blk, axis=1)
        iota = lax.broadcasted_iota(jnp.int32, (TN, CHUNK), 1)
        mi = jnp.min(jnp.where(blk == mv[:, None], iota, CHUNK), axis=1) + c * CHUNK
        take = mv < acc_v
        acc_i = jnp.where(take, mi, acc_i)
        acc_v = jnp.where(take, mv.astype(jnp.bfloat16).astype(jnp.float32), acc_v)
    return acc_i


def _vq_main(xm2_ref, w_ref, xsq_ref, wsq_ref, idx_ref, counts_ref):
    xm2 = xm2_ref[...]
    w = w_ref[...]
    mm = lax.dot_general(
        xm2, w, dimension_numbers=(((1,), (1,)), ((), ())),
        preferred_element_type=jnp.float32)
    d = (xsq_ref[...].reshape(TN, 1) + mm) + wsq_ref[...]
    idx = _argmin_rowwise(d)
    idx_ref[...] = idx.reshape(1, 1, TN)
    oh = (lax.broadcasted_iota(jnp.int32, (TN, K), 1)
          == idx.reshape(TN, 1)).astype(jnp.float32)
    part = jnp.sum(oh, axis=0).reshape(1, K)

    @pl.when(pl.program_id(0) == 0)
    def _init():
        counts_ref[...] = part

    @pl.when(pl.program_id(0) != 0)
    def _acc():
        counts_ref[...] = counts_ref[...] + part


DP = 128


def _sc_gather(w_hbm, idx_hbm, q_hbm, idx_v, rows_v, sem):
    wid = lax.axis_index("s") * NC + lax.axis_index("c")
    base = wid * BPW
    pltpu.sync_copy(idx_hbm.at[pl.ds(base, BPW)], idx_v)
    pltpu.async_copy(w_hbm.at[idx_v], rows_v, sem).wait()
    pltpu.sync_copy(rows_v, q_hbm.at[pl.ds(base, BPW)])


def _vq_stats(x_ref, q_ref, counts_ref, qst_ref, loss_ref, cl_ref, cbl_ref,
              perp_ref, usage_ref):
    x = x_ref[...]
    q = q_ref[...]
    qst_ref[...] = x + (q - x)
    msq = jnp.mean((q - x) ** 2)
    cl_ref[...] = msq.reshape(1, 1)
    cbl_ref[...] = msq.reshape(1, 1)
    loss_ref[...] = (msq + COMMIT * msq).reshape(1, 1)
    counts = counts_ref[...]
    p = counts / jnp.float32(N)
    ent = -jnp.sum(p * jnp.log(p + 1e-10))
    perp_ref[...] = jnp.exp(ent).reshape(1, 1)
    usage_ref[...] = jnp.mean((counts > 0).astype(jnp.float32)).reshape(1, 1)


@jax.jit
def kernel(inputs, weight):
    xsq = jnp.sum(inputs ** 2, axis=1, keepdims=True)
    wsq = jnp.sum(weight ** 2, axis=1)

    idx3, counts = pl.pallas_call(
        _vq_main,
        grid=(GRID,),
        in_specs=[
            pl.BlockSpec((TN, D), lambda i: (i, 0)),
            pl.BlockSpec((K, D), lambda i: (0, 0)),
            pl.BlockSpec((TN, 1), lambda i: (i, 0)),
            pl.BlockSpec((1, K), lambda i: (0, 0)),
        ],
        out_specs=[
            pl.BlockSpec((1, 1, TN), lambda i: (i, 0, 0)),
            pl.BlockSpec((1, K), lambda i: (0, 0)),
        ],
        out_shape=[
            jax.ShapeDtypeStruct((GRID, 1, TN), jnp.int32),
            jax.ShapeDtypeStruct((1, K), jnp.float32),
        ],
    )(-2.0 * inputs, weight, xsq, wsq.reshape(1, K))
    encoding_indices = idx3.reshape(N)

    mesh = plsc.VectorSubcoreMesh(core_axis_name="c", subcore_axis_name="s")
    wpad = jnp.pad(weight, ((0, 0), (0, DP - D)))
    qpad = pl.kernel(
        _sc_gather,
        mesh=mesh,
        out_type=jax.ShapeDtypeStruct((N, DP), jnp.float32),
        scratch_types=[
            pltpu.VMEM((BPW,), jnp.int32),
            pltpu.VMEM((BPW, DP), jnp.float32),
            pltpu.SemaphoreType.DMA,
        ],
    )(wpad, encoding_indices)
    quantized = qpad[:, :D]

    qst, loss, cl, cbl, perp, usage = pl.pallas_call(
        _vq_stats,
        out_shape=[
            jax.ShapeDtypeStruct((N, D), jnp.float32),
            jax.ShapeDtypeStruct((1, 1), jnp.float32),
            jax.ShapeDtypeStruct((1, 1), jnp.float32),
            jax.ShapeDtypeStruct((1, 1), jnp.float32),
            jax.ShapeDtypeStruct((1, 1), jnp.float32),
            jax.ShapeDtypeStruct((1, 1), jnp.float32),
        ],
    )(inputs, quantized, counts)

    return (qst, encoding_indices, loss[0, 0], cl[0, 0], cbl[0, 0],
            perp[0, 0], usage[0, 0])


# speedup vs baseline: 1.2739x; 1.1355x over previous
import functools

import jax
import jax.numpy as jnp
from jax import lax
from jax.experimental import pallas as pl
from jax.experimental.pallas import tpu as pltpu
from jax.experimental.pallas import tpu_sc as plsc

K = 8192
N = 8192
D = 32
TN = 256
GRID = N // TN
COMMIT = 0.25

CHUNK = 2048
NCHUNK = K // CHUNK

NC, NS, L = 2, 16, 16
NW = NC * NS
BPW = N // NW


def _argmin_rowwise(d):
    acc_v = jnp.full((TN,), jnp.inf, jnp.float32)
    acc_i = jnp.zeros((TN,), jnp.int32)
    for c in range(NCHUNK):
        blk = d[:, c * CHUNK:(c + 1) * CHUNK]
        mv = jnp.min(blk, axis=1)
        iota = lax.broadcasted_iota(jnp.int32, (TN, CHUNK), 1)
        mi = jnp.min(jnp.where(blk == mv[:, None], iota, CHUNK), axis=1) + c * CHUNK
        take = mv < acc_v
        acc_i = jnp.where(take, mi, acc_i)
        acc_v = jnp.where(take, mv.astype(jnp.bfloat16).astype(jnp.float32), acc_v)
    return acc_i


def _vq_main(xm2_ref, w_ref, xsq_ref, wsq_ref, idx_ref, counts_ref):
    xm2 = xm2_ref[...]
    w = w_ref[...]
    mm = lax.dot_general(
        xm2, w, dimension_numbers=(((1,), (1,)), ((), ())),
        preferred_element_type=jnp.float32)
    d = (xsq_ref[...].reshape(TN, 1) + mm) + wsq_ref[...]
    idx = _argmin_rowwise(d)
    idx_ref[...] = idx.reshape(1, 1, TN)
    oh = (lax.broadcasted_iota(jnp.int32, (TN, K), 1)
          == idx.reshape(TN, 1)).astype(jnp.float32)
    part = lax.dot_general(
        jnp.ones((1, TN), jnp.float32), oh,
        dimension_numbers=(((1,), (0,)), ((), ())),
        preferred_element_type=jnp.float32)

    @pl.when(pl.program_id(0) == 0)
    def _init():
        counts_ref[...] = part

    @pl.when(pl.program_id(0) != 0)
    def _acc():
        counts_ref[...] = counts_ref[...] + part


DP = 128


def _sc_gather(w_hbm, idx_hbm, q_hbm, idx_v, rows_v, sem):
    wid = lax.axis_index("s") * NC + lax.axis_index("c")
    base = wid * BPW
    pltpu.sync_copy(idx_hbm.at[pl.ds(base, BPW)], idx_v)
    pltpu.async_copy(w_hbm.at[idx_v], rows_v, sem).wait()
    pltpu.sync_copy(rows_v, q_hbm.at[pl.ds(base, BPW)])


def _vq_stats(x_ref, q_ref, counts_ref, qst_ref, loss_ref, cl_ref, cbl_ref,
              perp_ref, usage_ref):
    x = x_ref[...]
    q = q_ref[...]
    qst_ref[...] = x + (q - x)
    msq = jnp.mean((q - x) ** 2)
    cl_ref[...] = msq.reshape(1, 1)
    cbl_ref[...] = msq.reshape(1, 1)
    loss_ref[...] = (msq + COMMIT * msq).reshape(1, 1)
    counts = counts_ref[...]
    p = counts / jnp.float32(N)
    ent = -jnp.sum(p * jnp.log(p + 1e-10))
    perp_ref[...] = jnp.exp(ent).reshape(1, 1)
    usage_ref[...] = jnp.mean((counts > 0).astype(jnp.float32)).reshape(1, 1)


@jax.jit
def kernel(inputs, weight):
    xsq = jnp.sum(inputs ** 2, axis=1, keepdims=True)
    wsq = jnp.sum(weight ** 2, axis=1)

    idx3, counts = pl.pallas_call(
        _vq_main,
        grid=(GRID,),
        in_specs=[
            pl.BlockSpec((TN, D), lambda i: (i, 0)),
            pl.BlockSpec((K, D), lambda i: (0, 0)),
            pl.BlockSpec((TN, 1), lambda i: (i, 0)),
            pl.BlockSpec((1, K), lambda i: (0, 0)),
        ],
        out_specs=[
            pl.BlockSpec((1, 1, TN), lambda i: (i, 0, 0)),
            pl.BlockSpec((1, K), lambda i: (0, 0)),
        ],
        out_shape=[
            jax.ShapeDtypeStruct((GRID, 1, TN), jnp.int32),
            jax.ShapeDtypeStruct((1, K), jnp.float32),
        ],
    )(-2.0 * inputs, weight, xsq, wsq.reshape(1, K))
    encoding_indices = idx3.reshape(N)

    mesh = plsc.VectorSubcoreMesh(core_axis_name="c", subcore_axis_name="s")
    wpad = jnp.pad(weight, ((0, 0), (0, DP - D)))
    qpad = pl.kernel(
        _sc_gather,
        mesh=mesh,
        out_type=jax.ShapeDtypeStruct((N, DP), jnp.float32),
        scratch_types=[
            pltpu.VMEM((BPW,), jnp.int32),
            pltpu.VMEM((BPW, DP), jnp.float32),
            pltpu.SemaphoreType.DMA,
        ],
    )(wpad, encoding_indices)
    quantized = qpad[:, :D]

    qst, loss, cl, cbl, perp, usage = pl.pallas_call(
        _vq_stats,
        out_shape=[
            jax.ShapeDtypeStruct((N, D), jnp.float32),
            jax.ShapeDtypeStruct((1, 1), jnp.float32),
            jax.ShapeDtypeStruct((1, 1), jnp.float32),
            jax.ShapeDtypeStruct((1, 1), jnp.float32),
            jax.ShapeDtypeStruct((1, 1), jnp.float32),
            jax.ShapeDtypeStruct((1, 1), jnp.float32),
        ],
    )(inputs, quantized, counts)

    return (qst, encoding_indices, loss[0, 0], cl[0, 0], cbl[0, 0],
            perp[0, 0], usage[0, 0])


# speedup vs baseline: 1.3232x; 1.0387x over previous
import functools

import jax
import jax.numpy as jnp
from jax import lax
from jax.experimental import pallas as pl
from jax.experimental.pallas import tpu as pltpu
from jax.experimental.pallas import tpu_sc as plsc

K = 8192
N = 8192
D = 32
TN = 512
GRID = N // TN
COMMIT = 0.25

CHUNK = 2048
NCHUNK = K // CHUNK

NC, NS, L = 2, 16, 16
NW = NC * NS
BPW = N // NW


def _argmin_rowwise(d):
    acc_v = jnp.full((TN,), jnp.inf, jnp.float32)
    acc_i = jnp.zeros((TN,), jnp.int32)
    for c in range(NCHUNK):
        blk = d[:, c * CHUNK:(c + 1) * CHUNK]
        mv = jnp.min(blk, axis=1)
        iota = lax.broadcasted_iota(jnp.int32, (TN, CHUNK), 1)
        mi = jnp.min(jnp.where(blk == mv[:, None], iota, CHUNK), axis=1) + c * CHUNK
        take = mv < acc_v
        acc_i = jnp.where(take, mi, acc_i)
        acc_v = jnp.where(take, mv.astype(jnp.bfloat16).astype(jnp.float32), acc_v)
    return acc_i


def _vq_main(x_ref, w_ref, xsq_ref, wsq_ref, idx_ref, counts_ref):
    x = x_ref[...]
    w = w_ref[...]
    mm = lax.dot_general(
        x, w, dimension_numbers=(((1,), (1,)), ((), ())),
        preferred_element_type=jnp.float32)
    d = (xsq_ref[...].reshape(TN, 1) - (mm + mm)) + wsq_ref[...]
    idx = _argmin_rowwise(d)
    idx_ref[...] = idx.reshape(1, 1, TN)
    oh = (lax.broadcasted_iota(jnp.int32, (TN, K), 1)
          == idx.reshape(TN, 1)).astype(jnp.float32)
    part = lax.dot_general(
        jnp.ones((1, TN), jnp.float32), oh,
        dimension_numbers=(((1,), (0,)), ((), ())),
        preferred_element_type=jnp.float32)

    @pl.when(pl.program_id(0) == 0)
    def _init():
        counts_ref[...] = part

    @pl.when(pl.program_id(0) != 0)
    def _acc():
        counts_ref[...] = counts_ref[...] + part


DP = 128


def _sc_gather(w_hbm, idx_hbm, q_hbm, idx_v, rows_v, sem):
    wid = lax.axis_index("s") * NC + lax.axis_index("c")
    base = wid * BPW
    pltpu.sync_copy(idx_hbm.at[pl.ds(base, BPW)], idx_v)
    pltpu.async_copy(w_hbm.at[idx_v], rows_v, sem).wait()
    pltpu.sync_copy(rows_v, q_hbm.at[pl.ds(base, BPW)])


def _vq_stats(x_ref, q_ref, counts_ref, qst_ref, loss_ref, cl_ref, cbl_ref,
              perp_ref, usage_ref):
    x = x_ref[...]
    q = q_ref[...]
    qst_ref[...] = x + (q - x)
    msq = jnp.mean((q - x) ** 2)
    cl_ref[...] = msq.reshape(1, 1)
    cbl_ref[...] = msq.reshape(1, 1)
    loss_ref[...] = (msq + COMMIT * msq).reshape(1, 1)
    counts = counts_ref[...]
    p = counts / jnp.float32(N)
    ent = -jnp.sum(p * jnp.log(p + 1e-10))
    perp_ref[...] = jnp.exp(ent).reshape(1, 1)
    usage_ref[...] = jnp.mean((counts > 0).astype(jnp.float32)).reshape(1, 1)


@jax.jit
def kernel(inputs, weight):
    xsq = jnp.sum(inputs ** 2, axis=1, keepdims=True)
    wsq = jnp.sum(weight ** 2, axis=1)

    idx3, counts = pl.pallas_call(
        _vq_main,
        grid=(GRID,),
        in_specs=[
            pl.BlockSpec((TN, D), lambda i: (i, 0)),
            pl.BlockSpec((K, D), lambda i: (0, 0)),
            pl.BlockSpec((TN, 1), lambda i: (i, 0)),
            pl.BlockSpec((1, K), lambda i: (0, 0)),
        ],
        out_specs=[
            pl.BlockSpec((1, 1, TN), lambda i: (i, 0, 0)),
            pl.BlockSpec((1, K), lambda i: (0, 0)),
        ],
        out_shape=[
            jax.ShapeDtypeStruct((GRID, 1, TN), jnp.int32),
            jax.ShapeDtypeStruct((1, K), jnp.float32),
        ],
    )(inputs, weight, xsq, wsq.reshape(1, K))
    encoding_indices = idx3.reshape(N)

    mesh = plsc.VectorSubcoreMesh(core_axis_name="c", subcore_axis_name="s")
    wpad = jnp.pad(weight, ((0, 0), (0, DP - D)))
    qpad = pl.kernel(
        _sc_gather,
        mesh=mesh,
        out_type=jax.ShapeDtypeStruct((N, DP), jnp.float32),
        scratch_types=[
            pltpu.VMEM((BPW,), jnp.int32),
            pltpu.VMEM((BPW, DP), jnp.float32),
            pltpu.SemaphoreType.DMA,
        ],
    )(wpad, encoding_indices)
    quantized = qpad[:, :D]

    qst, loss, cl, cbl, perp, usage = pl.pallas_call(
        _vq_stats,
        out_shape=[
            jax.ShapeDtypeStruct((N, D), jnp.float32),
            jax.ShapeDtypeStruct((1, 1), jnp.float32),
            jax.ShapeDtypeStruct((1, 1), jnp.float32),
            jax.ShapeDtypeStruct((1, 1), jnp.float32),
            jax.ShapeDtypeStruct((1, 1), jnp.float32),
            jax.ShapeDtypeStruct((1, 1), jnp.float32),
        ],
    )(inputs, quantized, counts)

    return (qst, encoding_indices, loss[0, 0], cl[0, 0], cbl[0, 0],
            perp[0, 0], usage[0, 0])


# speedup vs baseline: 1.3257x; 1.0019x over previous
import functools

import jax
import jax.numpy as jnp
from jax import lax
from jax.experimental import pallas as pl
from jax.experimental.pallas import tpu as pltpu
from jax.experimental.pallas import tpu_sc as plsc

K = 8192
N = 8192
D = 32
TN = 512
GRID = N // TN
COMMIT = 0.25

CHUNK = 2048
NCHUNK = K // CHUNK

NC, NS, L = 2, 16, 16
NW = NC * NS
BPW = N // NW


def _argmin_rowwise(d):
    acc_v = jnp.full((TN,), jnp.inf, jnp.float32)
    acc_i = jnp.zeros((TN,), jnp.int32)
    for c in range(NCHUNK):
        blk = d[:, c * CHUNK:(c + 1) * CHUNK]
        mv = jnp.min(blk, axis=1)
        iota = lax.broadcasted_iota(jnp.int32, (TN, CHUNK), 1)
        mi = jnp.min(jnp.where(blk == mv[:, None], iota, CHUNK), axis=1) + c * CHUNK
        take = mv < acc_v
        acc_i = jnp.where(take, mi, acc_i)
        acc_v = jnp.where(take, mv.astype(jnp.bfloat16).astype(jnp.float32), acc_v)
    return acc_i


def _vq_main(x_ref, w_ref, xsq_ref, wsq_ref, idx_ref, counts_ref):
    x = x_ref[...]
    w = w_ref[...]
    mm = lax.dot_general(
        x, w, dimension_numbers=(((1,), (1,)), ((), ())),
        preferred_element_type=jnp.float32)
    d = (xsq_ref[...].reshape(TN, 1) - (mm + mm)) + wsq_ref[...]
    idx = _argmin_rowwise(d)
    idx_ref[...] = idx.reshape(1, 1, TN)
    oh = (lax.broadcasted_iota(jnp.int32, (TN, K), 1)
          == idx.reshape(TN, 1)).astype(jnp.float32)
    part = lax.dot_general(
        jnp.ones((1, TN), jnp.float32), oh,
        dimension_numbers=(((1,), (0,)), ((), ())),
        preferred_element_type=jnp.float32)

    @pl.when(pl.program_id(0) == 0)
    def _init():
        counts_ref[...] = part

    @pl.when(pl.program_id(0) != 0)
    def _acc():
        counts_ref[...] = counts_ref[...] + part


DP = 128


def _sc_gather(w_hbm, idx_hbm, q_hbm, idx_v, rows_v, sem):
    wid = lax.axis_index("s") * NC + lax.axis_index("c")
    base = wid * BPW
    pltpu.sync_copy(idx_hbm.at[pl.ds(base, BPW)], idx_v)
    pltpu.async_copy(w_hbm.at[idx_v], rows_v, sem).wait()
    pltpu.sync_copy(rows_v, q_hbm.at[pl.ds(base, BPW)])


def _vq_stats(x_ref, q_ref, counts_ref, qst_ref, loss_ref, cl_ref, cbl_ref,
              perp_ref, usage_ref):
    x = x_ref[...]
    q = q_ref[...].astype(jnp.bfloat16).astype(jnp.float32)
    qst_ref[...] = x + (q - x)
    msq = jnp.mean((q - x) ** 2)
    cl_ref[...] = msq.reshape(1, 1)
    cbl_ref[...] = msq.reshape(1, 1)
    loss_ref[...] = (msq + COMMIT * msq).reshape(1, 1)
    counts = counts_ref[...]
    p = counts / jnp.float32(N)
    ent = -jnp.sum(p * jnp.log(p + 1e-10))
    perp_ref[...] = jnp.exp(ent).reshape(1, 1)
    usage_ref[...] = jnp.mean((counts > 0).astype(jnp.float32)).reshape(1, 1)


@jax.jit
def kernel(inputs, weight):
    xsq = jnp.sum(inputs ** 2, axis=1, keepdims=True)
    wsq = jnp.sum(weight ** 2, axis=1)

    idx3, counts = pl.pallas_call(
        _vq_main,
        grid=(GRID,),
        in_specs=[
            pl.BlockSpec((TN, D), lambda i: (i, 0)),
            pl.BlockSpec((K, D), lambda i: (0, 0)),
            pl.BlockSpec((TN, 1), lambda i: (i, 0)),
            pl.BlockSpec((1, K), lambda i: (0, 0)),
        ],
        out_specs=[
            pl.BlockSpec((1, 1, TN), lambda i: (i, 0, 0)),
            pl.BlockSpec((1, K), lambda i: (0, 0)),
        ],
        out_shape=[
            jax.ShapeDtypeStruct((GRID, 1, TN), jnp.int32),
            jax.ShapeDtypeStruct((1, K), jnp.float32),
        ],
    )(inputs, weight, xsq, wsq.reshape(1, K))
    encoding_indices = idx3.reshape(N)

    mesh = plsc.VectorSubcoreMesh(core_axis_name="c", subcore_axis_name="s")
    wpad = jnp.pad(weight, ((0, 0), (0, DP - D)))
    qpad = pl.kernel(
        _sc_gather,
        mesh=mesh,
        out_type=jax.ShapeDtypeStruct((N, DP), jnp.float32),
        scratch_types=[
            pltpu.VMEM((BPW,), jnp.int32),
            pltpu.VMEM((BPW, DP), jnp.float32),
            pltpu.SemaphoreType.DMA,
        ],
    )(wpad, encoding_indices)
    quantized = qpad[:, :D]

    qst, loss, cl, cbl, perp, usage = pl.pallas_call(
        _vq_stats,
        out_shape=[
            jax.ShapeDtypeStruct((N, D), jnp.float32),
            jax.ShapeDtypeStruct((1, 1), jnp.float32),
            jax.ShapeDtypeStruct((1, 1), jnp.float32),
            jax.ShapeDtypeStruct((1, 1), jnp.float32),
            jax.ShapeDtypeStruct((1, 1), jnp.float32),
            jax.ShapeDtypeStruct((1, 1), jnp.float32),
        ],
    )(inputs, quantized, counts)

    return (qst, encoding_indices, loss[0, 0], cl[0, 0], cbl[0, 0],
            perp[0, 0], usage[0, 0])
